# Initial kernel scaffold; baseline (speedup 1.0000x reference)
import jax
import jax.numpy as jnp
from jax.experimental import pallas as pl


def kernel(dists, idx, radii, features):
    raise NotImplementedError("write your pallas kernel here")


# speedup vs baseline: 30.6572x; 30.6572x over previous
import functools

import jax
import jax.numpy as jnp
from jax import lax
from jax.experimental import pallas as pl
from jax.experimental.pallas import tpu as pltpu
from jax.experimental.pallas import tpu_sc as plsc

_B, _H, _W, _K, _P, _C = 2, 384, 384, 8, 100000, 32
_N = _B * _H * _W
_NW = 32
_PIX_PER_W = _N // _NW
_BLK = 64
_FRAG = _BLK * _K
_CHUNK = 128
_NCHUNK = _FRAG // _CHUNK
_NBLK = _PIX_PER_W // _BLK


def _body(dists_hbm, idx_hbm, radii_hbm, feat_hbm, out_hbm,
          radii_v, ibuf, dbuf, wbuf, rbuf, rows_v, obuf, sem):
    wid = lax.axis_index("c") * 16 + lax.axis_index("s")
    pltpu.sync_copy(radii_hbm, radii_v)

    lane = lax.iota(jnp.int32, 16)
    fio = lane * _K

    def block(b, carry):
        base_pix = wid * _PIX_PER_W + b * _BLK
        pltpu.sync_copy(idx_hbm.at[pl.ds(base_pix * _K, _FRAG)], ibuf)
        pltpu.sync_copy(dists_hbm.at[pl.ds(base_pix * _K, _FRAG)], dbuf)

        cps = [
            pltpu.async_copy(feat_hbm.at[ibuf.at[pl.ds(j * _CHUNK, _CHUNK)]],
                             rows_v.at[pl.ds(j * _CHUNK, _CHUNK)], sem)
            for j in range(_NCHUNK)
        ]
        for cp in cps:
            cp.wait()

        for g in range(_FRAG // 16):
            iv = ibuf[pl.ds(g * 16, 16)]
            r = plsc.load_gather(radii_v, [iv])
            w = 1.0 - dbuf[pl.ds(g * 16, 16)] / (r * r)
            wbuf[pl.ds(g * 16, 16)] = w

        for pg in range(_BLK // 16):
            s = plsc.load_gather(wbuf, [fio + pg * 128])
            for k in range(1, _K):
                s = s + plsc.load_gather(wbuf, [fio + (pg * 128 + k)])
            rbuf[pl.ds(pg * 16, 16)] = 1.0 / jnp.maximum(s, 1e-10)

        def pix(p, _):
            f0 = p * _K
            acc0 = jnp.zeros((16,), jnp.float32)
            acc1 = jnp.zeros((16,), jnp.float32)
            for k in range(_K):
                wspl = plsc.load_gather(wbuf, [jnp.full((16,), f0 + k, jnp.int32)])
                acc0 = acc0 + wspl * rows_v[f0 + k, pl.ds(0, 16)]
                acc1 = acc1 + wspl * rows_v[f0 + k, pl.ds(16, 16)]
            rspl = plsc.load_gather(rbuf, [jnp.full((16,), p, jnp.int32)])
            obuf[pl.ds(p * _C, 16)] = acc0 * rspl
            obuf[pl.ds(p * _C + 16, 16)] = acc1 * rspl
            return _

        lax.fori_loop(0, _BLK, pix, 0)
        pltpu.sync_copy(obuf, out_hbm.at[pl.ds(base_pix * _C, _BLK * _C)])
        return carry

    lax.fori_loop(0, _NBLK, block, 0)


@jax.jit
def _render(d_flat, idx2, radii, features):
    mesh = plsc.VectorSubcoreMesh(core_axis_name="c", subcore_axis_name="s")
    f = pl.kernel(
        _body,
        out_type=jax.ShapeDtypeStruct((_N * _C,), jnp.float32),
        mesh=mesh,
        scratch_types=[
            pltpu.VMEM((_P,), jnp.float32),
            pltpu.VMEM((_FRAG,), jnp.int32),
            pltpu.VMEM((_FRAG,), jnp.float32),
            pltpu.VMEM((_FRAG,), jnp.float32),
            pltpu.VMEM((_BLK,), jnp.float32),
            pltpu.VMEM((_FRAG, _C), jnp.float32),
            pltpu.VMEM((_BLK * _C,), jnp.float32),
            pltpu.SemaphoreType.DMA,
        ],
        compiler_params=pltpu.CompilerParams(
            needs_layout_passes=False, use_tc_tiling_on_sc=False),
    )
    return f(d_flat, idx2, radii, features)


def kernel(dists, idx, radii, features):
    d_flat = dists.reshape(_N * _K)
    idx_flat = idx.reshape(_N * _K)
    out = _render(d_flat, idx_flat, radii, features)
    return out.reshape(_B, _H, _W, _C)
